# baseline (speedup 1.0000x reference)
import jax
import jax.numpy as jnp
from jax import lax
from jax.experimental import pallas as pl
from jax.experimental.pallas import tpu as pltpu
from jax.experimental.pallas import tpu_sc as plsc

CLS = 1000
FEAT = 128
BATCH = 16384
NC, NS, LANES = 2, 16, 16
NW = NC * NS
ROWS_W = BATCH // NW
CHUNK = 128
NCHUNK = ROWS_W // CHUNK
BINS = 1024


def _rsqrt(x):
    i = plsc.bitcast(x, jnp.int32)
    i = jnp.int32(0x5F3759DF) - (i >> 1)
    y = plsc.bitcast(i, jnp.float32)
    for _ in range(3):
        y = y * (1.5 - 0.5 * x * y * y)
    return y


def _sc_body(xs_hbm, ys_hbm, ct_hbm, sw_hbm, nw_hbm,
             idx_v, xs_v0, xs_v1, cr_v0, cr_v1, bs_v, bn_v,
             sem_x0, sem_x1, sem_g0, sem_g1):
    xs_bufs = (xs_v0, xs_v1)
    cr_bufs = (cr_v0, cr_v1)
    sems_x = (sem_x0, sem_x1)
    sems_g = (sem_g0, sem_g1)
    wid = lax.axis_index("s") * NC + lax.axis_index("c")
    base = wid * ROWS_W
    zero16 = jnp.zeros((LANES,), jnp.float32)
    ones16 = jnp.ones((LANES,), jnp.float32)
    lane = lax.iota(jnp.int32, LANES)

    pltpu.sync_copy(ys_hbm.at[pl.ds(base, ROWS_W)], idx_v)

    def start(ck):
        b = ck % 2
        return (
            pltpu.async_copy(
                xs_hbm.at[pl.ds((base + ck * CHUNK) * FEAT, CHUNK * FEAT)],
                xs_bufs[b], sems_x[b]),
            pltpu.async_copy(ct_hbm.at[idx_v.at[pl.ds(ck * CHUNK, CHUNK)]],
                             cr_bufs[b], sems_g[b]),
        )

    inflight = {0: start(0)}

    def zero_bins(i, carry):
        bs_v[pl.ds(i * LANES, LANES)] = zero16
        bn_v[pl.ds(i * LANES, LANES)] = zero16
        return carry

    lax.fori_loop(0, BINS // LANES, zero_bins, 0, unroll=8)

    lane_j = [lane + j for j in range(LANES)]

    for ck in range(NCHUNK):
        if ck + 1 < NCHUNK:
            inflight[ck + 1] = start(ck + 1)
        cx, cg = inflight.pop(ck)
        cx.wait()
        cg.wait()
        xs_v = xs_bufs[ck % 2]
        cr_v = cr_bufs[ck % 2]

        def group_step(g, carry):
            rows = lane + g * LANES
            y16 = idx_v[pl.ds(ck * CHUNK + g * LANES, LANES)]
            fb = rows * FEAT + lane

            def acc_feature(xi, cv, fcarry):
                s1, d, s3 = fcarry
                x = plsc.load_gather(xs_v, [xi])
                c = plsc.load_gather(cr_v, [rows, cv])
                return (s1 + x * x, d + x * c, s3 + c * c)

            def block_step(i, fcarry):
                f0 = i * LANES
                for j in range(LANES):
                    fcarry = acc_feature(fb + (f0 + j), lane_j[j] + f0,
                                         fcarry)
                return fcarry

            fcarry = lax.fori_loop(0, FEAT // LANES - 1, block_step,
                                   (zero16, zero16, zero16))
            for j in range(LANES):
                cv = (lane_j[j] + (FEAT - LANES)) & (FEAT - 1)
                fcarry = acc_feature(rows * FEAT + cv, cv, fcarry)
            s1, d, s3 = fcarry
            inv = jnp.minimum(_rsqrt(s1), jnp.float32(1e12))
            d2 = s1 * inv * inv - 2.0 * (d * inv) + s3
            d2 = jnp.maximum(d2, 0.0)
            dist = d2 * _rsqrt(d2)
            plsc.addupdate_scatter(bs_v, [y16], dist)
            plsc.addupdate_scatter(bn_v, [y16], ones16)
            return carry

        lax.fori_loop(0, CHUNK // LANES, group_step, 0)

    pltpu.sync_copy(bs_v, sw_hbm.at[wid])
    pltpu.sync_copy(bn_v, nw_hbm.at[wid])


_sc_hist = pl.kernel(
    _sc_body,
    out_type=(jax.ShapeDtypeStruct((NW, BINS), jnp.float32),
              jax.ShapeDtypeStruct((NW, BINS), jnp.float32)),
    mesh=plsc.VectorSubcoreMesh(core_axis_name="c", subcore_axis_name="s",
                                num_cores=NC, num_subcores=NS),
    scratch_types=[
        pltpu.VMEM((ROWS_W,), jnp.int32),
        pltpu.VMEM((CHUNK * FEAT,), jnp.float32),
        pltpu.VMEM((CHUNK * FEAT,), jnp.float32),
        pltpu.VMEM((CHUNK, FEAT), jnp.float32),
        pltpu.VMEM((CHUNK, FEAT), jnp.float32),
        pltpu.VMEM((BINS,), jnp.float32),
        pltpu.VMEM((BINS,), jnp.float32),
        pltpu.SemaphoreType.DMA,
        pltpu.SemaphoreType.DMA,
        pltpu.SemaphoreType.DMA,
        pltpu.SemaphoreType.DMA,
    ],
    compiler_params=pltpu.CompilerParams(needs_layout_passes=False),
)


def _tc_body(s_ref, n_ref, o_ref):
    s = jnp.sum(s_ref[...], axis=0, keepdims=True)
    n = jnp.sum(n_ref[...], axis=0, keepdims=True)
    o_ref[...] = jnp.sum(jnp.where(n > 0.0, s / n, 0.0),
                         axis=1, keepdims=True)


def kernel(xs, ys, center):
    sw, nw = _sc_hist(xs.reshape(-1), ys.astype(jnp.int32), center)
    out = pl.pallas_call(
        _tc_body,
        out_shape=jax.ShapeDtypeStruct((1, 1), jnp.float32),
    )(sw, nw)
    return out[0, 0]


# speedup vs baseline: 1.0651x; 1.0651x over previous
import jax
import jax.numpy as jnp
from jax import lax
from jax.experimental import pallas as pl
from jax.experimental.pallas import tpu as pltpu
from jax.experimental.pallas import tpu_sc as plsc

CLS = 1000
FEAT = 128
BATCH = 16384
NC, NS, LANES = 2, 16, 16
NW = NC * NS
ROWS_W = BATCH // NW
CHUNK = 128
NCHUNK = ROWS_W // CHUNK
BINS = 1024


def _rsqrt(x):
    i = plsc.bitcast(x, jnp.int32)
    i = jnp.int32(0x5F3759DF) - (i >> 1)
    y = plsc.bitcast(i, jnp.float32)
    for _ in range(3):
        y = y * (1.5 - 0.5 * x * y * y)
    return y


def _sc_body(xs_hbm, ys_hbm, ct_hbm, sw_hbm, nw_hbm,
             idx_v, xs_v0, xs_v1, cr_v0, cr_v1, bs_v, bn_v,
             sem_x0, sem_x1, sem_g0, sem_g1):
    xs_bufs = (xs_v0, xs_v1)
    cr_bufs = (cr_v0, cr_v1)
    sems_x = (sem_x0, sem_x1)
    sems_g = (sem_g0, sem_g1)
    wid = lax.axis_index("s") * NC + lax.axis_index("c")
    base = wid * ROWS_W
    zero16 = jnp.zeros((LANES,), jnp.float32)
    ones16 = jnp.ones((LANES,), jnp.float32)
    lane = lax.iota(jnp.int32, LANES)

    pltpu.sync_copy(ys_hbm.at[pl.ds(base, ROWS_W)], idx_v)

    def start(ck):
        b = ck % 2
        return (
            pltpu.async_copy(
                xs_hbm.at[pl.ds((base + ck * CHUNK) * FEAT, CHUNK * FEAT)],
                xs_bufs[b], sems_x[b]),
            pltpu.async_copy(ct_hbm.at[idx_v.at[pl.ds(ck * CHUNK, CHUNK)]],
                             cr_bufs[b], sems_g[b]),
        )

    inflight = {0: start(0)}

    def zero_bins(i, carry):
        bs_v[pl.ds(i * LANES, LANES)] = zero16
        bn_v[pl.ds(i * LANES, LANES)] = zero16
        return carry

    lax.fori_loop(0, BINS // LANES, zero_bins, 0, unroll=8)

    for ck in range(NCHUNK):
        if ck + 1 < NCHUNK:
            inflight[ck + 1] = start(ck + 1)
        cx, cg = inflight.pop(ck)
        cx.wait()
        cg.wait()
        xs_v = xs_bufs[ck % 2]
        cr_v = cr_bufs[ck % 2]

        def group_step(g, carry):
            rows = lane + g * LANES
            y16 = idx_v[pl.ds(ck * CHUNK + g * LANES, LANES)]

            def feat_step(f, fcarry):
                s1, d, s3 = fcarry
                fv = (lane + f) & (FEAT - 1)
                x = plsc.load_gather(xs_v, [rows * FEAT + fv])
                c = plsc.load_gather(cr_v, [rows, fv])
                return (s1 + x * x, d + x * c, s3 + c * c)

            s1, d, s3 = lax.fori_loop(0, FEAT, feat_step,
                                      (zero16, zero16, zero16), unroll=16)
            inv = jnp.minimum(_rsqrt(s1), jnp.float32(1e12))
            d2 = s1 * inv * inv - 2.0 * (d * inv) + s3
            d2 = jnp.maximum(d2, 0.0)
            dist = d2 * _rsqrt(d2)
            plsc.addupdate_scatter(bs_v, [y16], dist)
            plsc.addupdate_scatter(bn_v, [y16], ones16)
            return carry

        lax.fori_loop(0, CHUNK // LANES, group_step, 0)

    pltpu.sync_copy(bs_v, sw_hbm.at[wid])
    pltpu.sync_copy(bn_v, nw_hbm.at[wid])


_sc_hist = pl.kernel(
    _sc_body,
    out_type=(jax.ShapeDtypeStruct((NW, BINS), jnp.float32),
              jax.ShapeDtypeStruct((NW, BINS), jnp.float32)),
    mesh=plsc.VectorSubcoreMesh(core_axis_name="c", subcore_axis_name="s",
                                num_cores=NC, num_subcores=NS),
    scratch_types=[
        pltpu.VMEM((ROWS_W,), jnp.int32),
        pltpu.VMEM((CHUNK * FEAT,), jnp.float32),
        pltpu.VMEM((CHUNK * FEAT,), jnp.float32),
        pltpu.VMEM((CHUNK, FEAT), jnp.float32),
        pltpu.VMEM((CHUNK, FEAT), jnp.float32),
        pltpu.VMEM((BINS,), jnp.float32),
        pltpu.VMEM((BINS,), jnp.float32),
        pltpu.SemaphoreType.DMA,
        pltpu.SemaphoreType.DMA,
        pltpu.SemaphoreType.DMA,
        pltpu.SemaphoreType.DMA,
    ],
    compiler_params=pltpu.CompilerParams(needs_layout_passes=False),
)


def _tc_body(s_ref, n_ref, o_ref):
    s = jnp.sum(s_ref[...], axis=0, keepdims=True)
    n = jnp.sum(n_ref[...], axis=0, keepdims=True)
    o_ref[...] = jnp.sum(jnp.where(n > 0.0, s / n, 0.0),
                         axis=1, keepdims=True)


def kernel(xs, ys, center):
    sw, nw = _sc_hist(xs.reshape(-1), ys.astype(jnp.int32), center)
    out = pl.pallas_call(
        _tc_body,
        out_shape=jax.ShapeDtypeStruct((1, 1), jnp.float32),
    )(sw, nw)
    return out[0, 0]
